# baseline (speedup 1.0000x reference)
import functools

import jax
import jax.numpy as jnp
from jax import lax
from jax.experimental import pallas as pl
from jax.experimental.pallas import tpu as pltpu
from jax.experimental.pallas import tpu_sc as plsc

BATCH = 16384
EMBED = 32
MLP_DIM = 128
_PACK = MLP_DIM // EMBED

_NC = 2
_NS = 16
_NW = _NC * _NS
_BPW = BATCH // _NW
_CHUNK = 128
_NCHUNK = _BPW // _CHUNK
_GCHUNK = 64
_NGCHUNK = _BPW // _GCHUNK


def _sc_gather_body(user_hbm, item_hbm, mu_hbm, mi_hbm, gu2_hbm, gi2_hbm,
                    out_u_hbm, out_i_hbm, out_g_hbm,
                    idx_u, idx_i, idx_qu, idx_qi,
                    rows_u0, rows_u1, rows_i0, rows_i1,
                    gbuf_u0, gbuf_u1, gbuf_i0, gbuf_i1, pbuf0, pbuf1,
                    sem_u0, sem_u1, sem_i0, sem_i1,
                    sem_gu0, sem_gu1, sem_gi0, sem_gi1):
    wid = lax.axis_index("s") * _NC + lax.axis_index("c")
    base = wid * _BPW
    rows_u = (rows_u0, rows_u1)
    rows_i = (rows_i0, rows_i1)
    sems_u = (sem_u0, sem_u1)
    sems_i = (sem_i0, sem_i1)
    gbuf_u = (gbuf_u0, gbuf_u1)
    gbuf_i = (gbuf_i0, gbuf_i1)
    pbuf = (pbuf0, pbuf1)
    sems_gu = (sem_gu0, sem_gu1)
    sems_gi = (sem_gi0, sem_gi1)

    for c in range(_NCHUNK):
        off = base + c * _CHUNK
        pltpu.sync_copy(user_hbm.at[pl.ds(off, _CHUNK)], idx_u.at[c])
        pltpu.sync_copy(item_hbm.at[pl.ds(off, _CHUNK)], idx_i.at[c])

    def _shift(g, _):
        c = lax.shift_right_logical(g, 3)
        mo = lax.bitwise_and(g, 7) * 16
        idx_qu[c, pl.ds(mo, 16)] = lax.shift_right_logical(
            idx_u[c, pl.ds(mo, 16)], 2)
        idx_qi[c, pl.ds(mo, 16)] = lax.shift_right_logical(
            idx_i[c, pl.ds(mo, 16)], 2)
        return _

    lax.fori_loop(0, _BPW // 16, _shift, 0, unroll=4)

    mh = {}
    gh = {}

    def _fire_mlp(c):
        s = c % 2
        mh[c] = (
            pltpu.async_copy(mu_hbm.at[idx_u.at[c]], rows_u[s], sems_u[s]),
            pltpu.async_copy(mi_hbm.at[idx_i.at[c]], rows_i[s], sems_i[s]),
        )

    def _fire_gmf(gc):
        s = gc % 2
        c, r0 = divmod(gc * _GCHUNK, _CHUNK)
        gh[gc] = (
            pltpu.async_copy(gu2_hbm.at[idx_qu.at[c, pl.ds(r0, _GCHUNK)]],
                             gbuf_u[s], sems_gu[s]),
            pltpu.async_copy(gi2_hbm.at[idx_qi.at[c, pl.ds(r0, _GCHUNK)]],
                             gbuf_i[s], sems_gi[s]),
        )

    _fire_mlp(0)
    _fire_mlp(1)
    _fire_gmf(0)

    for gc in range(_NGCHUNK):
        s = gc % 2
        if gc + 1 < _NGCHUNK:
            _fire_gmf(gc + 1)
        if gc % 2 == 0:
            c = gc // 2
            hu, hi = mh.pop(c)
            off = base + c * _CHUNK
            hu.wait()
            pltpu.sync_copy(rows_u[c % 2], out_u_hbm.at[pl.ds(off, _CHUNK)])
            hi.wait()
            pltpu.sync_copy(rows_i[c % 2], out_i_hbm.at[pl.ds(off, _CHUNK)])
            if c + 2 < _NCHUNK:
                _fire_mlp(c + 2)
        hgu, hgi = gh.pop(gc)
        hgu.wait()
        hgi.wait()
        mc, mo = divmod(gc * _GCHUNK, _CHUNK)

        def _prod(g, _):
            vu = idx_u[mc, pl.ds(mo + g * 16, 16)]
            vi = idx_i[mc, pl.ds(mo + g * 16, 16)]
            for j in range(16):
                r = g * 16 + j
                cu = lax.bitwise_and(vu[j], _PACK - 1) * EMBED
                ci = lax.bitwise_and(vi[j], _PACK - 1) * EMBED
                a = (gbuf_u[s][r, pl.ds(cu, 16)]
                     * gbuf_i[s][r, pl.ds(ci, 16)])
                b = (gbuf_u[s][r, pl.ds(cu + 16, 16)]
                     * gbuf_i[s][r, pl.ds(ci + 16, 16)])
                pbuf[s][r, pl.ds(0, 16)] = a
                pbuf[s][r, pl.ds(16, 16)] = b
            return _

        lax.fori_loop(0, _GCHUNK // 16, _prod, 0)
        pltpu.sync_copy(pbuf[s],
                        out_g_hbm.at[pl.ds(base + gc * _GCHUNK, _GCHUNK)])


_sc_gather = functools.partial(
    pl.kernel,
    out_type=(
        jax.ShapeDtypeStruct((BATCH, MLP_DIM), jnp.float32),
        jax.ShapeDtypeStruct((BATCH, MLP_DIM), jnp.float32),
        jax.ShapeDtypeStruct((BATCH, EMBED), jnp.float32),
    ),
    mesh=plsc.VectorSubcoreMesh(core_axis_name="c", subcore_axis_name="s",
                                num_cores=_NC, num_subcores=_NS),
    scratch_types=[
        pltpu.VMEM((_NCHUNK, _CHUNK), jnp.int32),
        pltpu.VMEM((_NCHUNK, _CHUNK), jnp.int32),
        pltpu.VMEM((_NCHUNK, _CHUNK), jnp.int32),
        pltpu.VMEM((_NCHUNK, _CHUNK), jnp.int32),
        pltpu.VMEM((_CHUNK, MLP_DIM), jnp.float32),
        pltpu.VMEM((_CHUNK, MLP_DIM), jnp.float32),
        pltpu.VMEM((_CHUNK, MLP_DIM), jnp.float32),
        pltpu.VMEM((_CHUNK, MLP_DIM), jnp.float32),
        pltpu.VMEM((_GCHUNK, MLP_DIM), jnp.float32),
        pltpu.VMEM((_GCHUNK, MLP_DIM), jnp.float32),
        pltpu.VMEM((_GCHUNK, MLP_DIM), jnp.float32),
        pltpu.VMEM((_GCHUNK, MLP_DIM), jnp.float32),
        pltpu.VMEM((_GCHUNK, EMBED), jnp.float32),
        pltpu.VMEM((_GCHUNK, EMBED), jnp.float32),
        pltpu.SemaphoreType.DMA,
        pltpu.SemaphoreType.DMA,
        pltpu.SemaphoreType.DMA,
        pltpu.SemaphoreType.DMA,
        pltpu.SemaphoreType.DMA,
        pltpu.SemaphoreType.DMA,
        pltpu.SemaphoreType.DMA,
        pltpu.SemaphoreType.DMA,
    ],
)(_sc_gather_body)


_BLK = 2048


def _tc_dense_body(u_ref, i_ref, g_ref, w0u_ref, w0i_ref, b0_ref,
                   w1_ref, b1_ref, w2_ref, b2_ref, wpg_ref, wpx_ref,
                   bp_ref, out_ref):
    dot = functools.partial(
        jax.lax.dot_general,
        dimension_numbers=(((1,), (0,)), ((), ())),
        preferred_element_type=jnp.float32,
        precision=jax.lax.Precision.DEFAULT,
    )
    x = dot(u_ref[...], w0u_ref[...]) + dot(i_ref[...], w0i_ref[...])
    x = jnp.maximum(x + b0_ref[...], 0.0)
    x = jnp.maximum(dot(x, w1_ref[...]) + b1_ref[...], 0.0)
    x = jnp.maximum(dot(x, w2_ref[...]) + b2_ref[...], 0.0)
    pred = (jnp.sum(g_ref[...] * wpg_ref[...], axis=-1, keepdims=True)
            + jnp.sum(x * wpx_ref[...], axis=-1, keepdims=True)
            + bp_ref[...])
    out_ref[...] = pred


def kernel(user, item, gmf_user_w, gmf_item_w, mlp_user_w, mlp_item_w,
           W0, b0, W1, b1, W2, b2, Wp, bp):
    gu2 = gmf_user_w.reshape(-1, MLP_DIM)
    gi2 = gmf_item_w.reshape(-1, MLP_DIM)
    u_rows, i_rows, g_rows = _sc_gather(
        user, item, mlp_user_w, mlp_item_w, gu2, gi2)

    w0t = W0.T
    w0u = w0t[:MLP_DIM]
    w0i = w0t[MLP_DIM:]
    w1t = W1.T
    w2t = W2.T
    wpg = Wp[:, :EMBED]
    wpx = Wp[:, EMBED:]

    nblk = BATCH // _BLK
    full = lambda s: pl.BlockSpec(s, lambda n: (0, 0))
    pred = pl.pallas_call(
        _tc_dense_body,
        grid=(nblk,),
        in_specs=[
            pl.BlockSpec((_BLK, MLP_DIM), lambda n: (n, 0)),
            pl.BlockSpec((_BLK, MLP_DIM), lambda n: (n, 0)),
            pl.BlockSpec((_BLK, EMBED), lambda n: (n, 0)),
            full((MLP_DIM, MLP_DIM)),
            full((MLP_DIM, MLP_DIM)),
            full((1, MLP_DIM)),
            full((MLP_DIM, 64)),
            full((1, 64)),
            full((64, EMBED)),
            full((1, EMBED)),
            full((1, EMBED)),
            full((1, EMBED)),
            full((1, 1)),
        ],
        out_specs=pl.BlockSpec((_BLK, 1), lambda n: (n, 0)),
        out_shape=jax.ShapeDtypeStruct((BATCH, 1), jnp.float32),
    )(u_rows, i_rows, g_rows, w0u, w0i, b0.reshape(1, -1),
      w1t, b1.reshape(1, -1), w2t, b2.reshape(1, -1), wpg, wpx,
      bp.reshape(1, 1))
    return pred.reshape(-1)


# speedup vs baseline: 1.5369x; 1.5369x over previous
import functools

import jax
import jax.numpy as jnp
from jax import lax
from jax.experimental import pallas as pl
from jax.experimental.pallas import tpu as pltpu
from jax.experimental.pallas import tpu_sc as plsc

BATCH = 16384
EMBED = 32
MLP_DIM = 128

_NC = 2
_NS = 16
_NW = _NC * _NS
_BPW = BATCH // _NW
_CHUNK = 128
_NCHUNK = _BPW // _CHUNK


_GCHUNK = 64
_NGCHUNK = _BPW // _GCHUNK


def _sc_gather_body(user_hbm, item_hbm, mu_hbm, mi_hbm, gu_hbm, gi_hbm,
                    out_u_hbm, out_i_hbm, out_g_hbm,
                    idx_u, idx_i, rows_u0, rows_u1, rows_i0, rows_i1,
                    gbuf_u0, gbuf_u1, gbuf_i0, gbuf_i1,
                    sem_u0, sem_u1, sem_i0, sem_i1,
                    sem_gu0, sem_gu1, sem_gi0, sem_gi1):
    wid = lax.axis_index("s") * _NC + lax.axis_index("c")
    base = wid * _BPW
    rows_u = (rows_u0, rows_u1)
    rows_i = (rows_i0, rows_i1)
    sems_u = (sem_u0, sem_u1)
    sems_i = (sem_i0, sem_i1)
    gbuf_u = (gbuf_u0, gbuf_u1)
    gbuf_i = (gbuf_i0, gbuf_i1)
    sems_gu = (sem_gu0, sem_gu1)
    sems_gi = (sem_gi0, sem_gi1)

    for c in range(_NCHUNK):
        off = base + c * _CHUNK
        pltpu.sync_copy(user_hbm.at[pl.ds(off, _CHUNK)], idx_u.at[c])
        pltpu.sync_copy(item_hbm.at[pl.ds(off, _CHUNK)], idx_i.at[c])

    mlp_h = {}

    def _fire_mlp(c):
        mlp_h[c] = (
            pltpu.async_copy(mu_hbm.at[idx_u.at[c]], rows_u[c % 2],
                             sems_u[c % 2]),
            pltpu.async_copy(mi_hbm.at[idx_i.at[c]], rows_i[c % 2],
                             sems_i[c % 2]),
        )

    def _fire_gmf(gc):
        slot = gc % 2

        def _grp(g, _):
            mc = lax.shift_right_logical(gc * _GCHUNK + g * 16, 7)
            mo = lax.bitwise_and(gc * _GCHUNK + g * 16, 127)
            vu = idx_u[mc, pl.ds(mo, 16)]
            vi = idx_i[mc, pl.ds(mo, 16)]
            for j in range(16):
                r = g * 16 + j
                pltpu.async_copy(gu_hbm.at[vu[j]], gbuf_u[slot].at[r],
                                 sems_gu[slot])
                pltpu.async_copy(gi_hbm.at[vi[j]], gbuf_i[slot].at[r],
                                 sems_gi[slot])
            return _

        lax.fori_loop(0, _GCHUNK // 16, _grp, 0)

    _fire_mlp(0)
    _fire_mlp(1)
    _fire_gmf(0)

    for gc in range(_NGCHUNK):
        slot = gc % 2
        if gc + 1 < _NGCHUNK:
            _fire_gmf(gc + 1)
        if gc % 2 == 0:
            c = gc // 2
            hu, hi = mlp_h.pop(c)
            off = base + c * _CHUNK
            hu.wait()
            pltpu.sync_copy(rows_u[c % 2], out_u_hbm.at[pl.ds(off, _CHUNK)])
            hi.wait()
            pltpu.sync_copy(rows_i[c % 2], out_i_hbm.at[pl.ds(off, _CHUNK)])
            if c + 2 < _NCHUNK:
                _fire_mlp(c + 2)
        pltpu.make_async_copy(gu_hbm.at[pl.ds(0, _GCHUNK)], gbuf_u[slot],
                              sems_gu[slot]).wait()
        pltpu.make_async_copy(gu_hbm.at[pl.ds(0, _GCHUNK)], gbuf_i[slot],
                              sems_gi[slot]).wait()

        def _prod(r, _):
            a = gbuf_u[slot][r, pl.ds(0, 16)] * gbuf_i[slot][r, pl.ds(0, 16)]
            b = gbuf_u[slot][r, pl.ds(16, 16)] * gbuf_i[slot][r, pl.ds(16, 16)]
            gbuf_u[slot][r, pl.ds(0, 16)] = a
            gbuf_u[slot][r, pl.ds(16, 16)] = b
            return _

        lax.fori_loop(0, _GCHUNK, _prod, 0, unroll=4)
        pltpu.sync_copy(gbuf_u[slot],
                        out_g_hbm.at[pl.ds(base + gc * _GCHUNK, _GCHUNK)])


_sc_gather = functools.partial(
    pl.kernel,
    out_type=(
        jax.ShapeDtypeStruct((BATCH, MLP_DIM), jnp.float32),
        jax.ShapeDtypeStruct((BATCH, MLP_DIM), jnp.float32),
        jax.ShapeDtypeStruct((BATCH, EMBED), jnp.float32),
    ),
    mesh=plsc.VectorSubcoreMesh(core_axis_name="c", subcore_axis_name="s",
                                num_cores=_NC, num_subcores=_NS),
    scratch_types=[
        pltpu.VMEM((_NCHUNK, _CHUNK), jnp.int32),
        pltpu.VMEM((_NCHUNK, _CHUNK), jnp.int32),
        pltpu.VMEM((_CHUNK, MLP_DIM), jnp.float32),
        pltpu.VMEM((_CHUNK, MLP_DIM), jnp.float32),
        pltpu.VMEM((_CHUNK, MLP_DIM), jnp.float32),
        pltpu.VMEM((_CHUNK, MLP_DIM), jnp.float32),
        pltpu.VMEM((_GCHUNK, EMBED), jnp.float32),
        pltpu.VMEM((_GCHUNK, EMBED), jnp.float32),
        pltpu.VMEM((_GCHUNK, EMBED), jnp.float32),
        pltpu.VMEM((_GCHUNK, EMBED), jnp.float32),
        pltpu.SemaphoreType.DMA,
        pltpu.SemaphoreType.DMA,
        pltpu.SemaphoreType.DMA,
        pltpu.SemaphoreType.DMA,
        pltpu.SemaphoreType.DMA,
        pltpu.SemaphoreType.DMA,
        pltpu.SemaphoreType.DMA,
        pltpu.SemaphoreType.DMA,
    ],
)(_sc_gather_body)


_BLK = 2048


def _tc_dense_body(u_ref, i_ref, g_ref, w0u_ref, w0i_ref, b0_ref,
                   w1_ref, b1_ref, w2_ref, b2_ref, wpg_ref, wpx_ref,
                   bp_ref, out_ref):
    dot = functools.partial(
        jax.lax.dot_general,
        dimension_numbers=(((1,), (0,)), ((), ())),
        preferred_element_type=jnp.float32,
        precision=jax.lax.Precision.DEFAULT,
    )
    x = dot(u_ref[...], w0u_ref[...]) + dot(i_ref[...], w0i_ref[...])
    x = jnp.maximum(x + b0_ref[...], 0.0)
    x = jnp.maximum(dot(x, w1_ref[...]) + b1_ref[...], 0.0)
    x = jnp.maximum(dot(x, w2_ref[...]) + b2_ref[...], 0.0)
    pred = (jnp.sum(g_ref[...] * wpg_ref[...], axis=-1, keepdims=True)
            + jnp.sum(x * wpx_ref[...], axis=-1, keepdims=True)
            + bp_ref[...])
    out_ref[...] = pred


def kernel(user, item, gmf_user_w, gmf_item_w, mlp_user_w, mlp_item_w,
           W0, b0, W1, b1, W2, b2, Wp, bp):
    u_rows, i_rows, g_rows = _sc_gather(
        user, item, mlp_user_w, mlp_item_w, gmf_user_w, gmf_item_w)

    w0t = W0.T
    w0u = w0t[:MLP_DIM]
    w0i = w0t[MLP_DIM:]
    w1t = W1.T
    w2t = W2.T
    wpg = Wp[:, :EMBED]
    wpx = Wp[:, EMBED:]

    nblk = BATCH // _BLK
    full = lambda s: pl.BlockSpec(s, lambda n: (0, 0))
    pred = pl.pallas_call(
        _tc_dense_body,
        grid=(nblk,),
        in_specs=[
            pl.BlockSpec((_BLK, MLP_DIM), lambda n: (n, 0)),
            pl.BlockSpec((_BLK, MLP_DIM), lambda n: (n, 0)),
            pl.BlockSpec((_BLK, EMBED), lambda n: (n, 0)),
            full((MLP_DIM, MLP_DIM)),
            full((MLP_DIM, MLP_DIM)),
            full((1, MLP_DIM)),
            full((MLP_DIM, 64)),
            full((1, 64)),
            full((64, EMBED)),
            full((1, EMBED)),
            full((1, EMBED)),
            full((1, EMBED)),
            full((1, 1)),
        ],
        out_specs=pl.BlockSpec((_BLK, 1), lambda n: (n, 0)),
        out_shape=jax.ShapeDtypeStruct((BATCH, 1), jnp.float32),
    )(u_rows, i_rows, g_rows, w0u, w0i, b0.reshape(1, -1),
      w1t, b1.reshape(1, -1), w2t, b2.reshape(1, -1), wpg, wpx,
      bp.reshape(1, 1))
    return pred.reshape(-1)
